# baseline (speedup 1.0000x reference)
import functools

import jax
import jax.numpy as jnp
from jax import lax
from jax.experimental import pallas as pl
from jax.experimental.pallas import tpu as pltpu
from jax.experimental.pallas import tpu_sc as plsc

B = 1024
ED = 16
VD = 64
NC = 2
NS = 16
NW = NC * NS
BPW = B // NW
BLK = 128
NBLK = B // BLK
BIAS_ROWS = 7813


def _sc_bias_gather(pos_items, neg_items, bias2):
    mesh = plsc.VectorSubcoreMesh(core_axis_name="c", subcore_axis_name="s")
    out_type = (
        jax.ShapeDtypeStruct((B,), jnp.float32),
        jax.ShapeDtypeStruct((B,), jnp.float32),
    )
    scratch_types = [
        pltpu.VMEM((BPW,), jnp.int32),
        pltpu.VMEM((BPW,), jnp.int32),
        pltpu.VMEM((BPW,), jnp.int32),
        pltpu.VMEM((BPW,), jnp.int32),
        pltpu.VMEM((BPW, 128), jnp.float32),
        pltpu.VMEM((BPW, 128), jnp.float32),
        pltpu.VMEM((BPW,), jnp.float32),
        pltpu.VMEM((BPW,), jnp.float32),
        pltpu.SemaphoreType.DMA,
        pltpu.SemaphoreType.DMA,
    ]

    @functools.partial(pl.kernel, mesh=mesh, out_type=out_type,
                       scratch_types=scratch_types,
                       compiler_params=pltpu.CompilerParams(
                           needs_layout_passes=False))
    def k(pos_h, neg_h, ib_t, pb_o, nb_o,
          p_i, n_i, pbg, nbg, pb_g, nb_g, pb_c, nb_c, gsem, osem):
        wid = lax.axis_index("s") * NC + lax.axis_index("c")
        base = wid * BPW
        pltpu.sync_copy(pos_h.at[pl.ds(base, BPW)], p_i)
        pltpu.sync_copy(neg_h.at[pl.ds(base, BPW)], n_i)
        for gidx in range(BPW // 16):
            sl = pl.ds(gidx * 16, 16)
            pbg[sl] = lax.shift_right_logical(p_i[sl], 7)
            nbg[sl] = lax.shift_right_logical(n_i[sl], 7)
        g1 = pltpu.async_copy(ib_t.at[pbg], pb_g, gsem)
        g2 = pltpu.async_copy(ib_t.at[nbg], nb_g, gsem)
        g1.wait()
        g2.wait()
        for gidx in range(BPW // 16):
            rows16 = pl.ds(gidx * 16, 16)
            rows = lax.iota(jnp.int32, 16) + gidx * 16
            pb_c[rows16] = plsc.load_gather(pb_g, [rows, p_i[rows16] & 127])
            nb_c[rows16] = plsc.load_gather(nb_g, [rows, n_i[rows16] & 127])
        s1 = pltpu.async_copy(pb_c, pb_o.at[pl.ds(base, BPW)], osem)
        s2 = pltpu.async_copy(nb_c, nb_o.at[pl.ds(base, BPW)], osem)
        s1.wait()
        s2.wait()

    return k(pos_items, neg_items, bias2)


def _tc_gather(users, pos_items, neg_items, user_emb, item_emb, uv_emb,
               v_feat):
    def body(users_s, pos_s, neg_s, ue_t, ie_t, uv_t, vf_t,
             ue_o, pe_o, ne_o, uve_o, pf_o, nf_o,
             s0, s1, s2, s3, s4, s5):
        def lp(i, carry):
            u4 = (users_s[i] >> 2) * 0
            p4 = (pos_s[i] >> 2) * 0
            n4 = (neg_s[i] >> 2) * 0
            blk = pl.ds(i * 4, 4)
            pltpu.make_async_copy(ue_t.at[pl.ds(u4, 4)], ue_o.at[blk],
                                  s0).start()
            pltpu.make_async_copy(ie_t.at[pl.ds(p4, 4)], pe_o.at[blk],
                                  s1).start()
            pltpu.make_async_copy(ie_t.at[pl.ds(n4, 4)], ne_o.at[blk],
                                  s2).start()
            pltpu.make_async_copy(uv_t.at[pl.ds(u4, 4)], uve_o.at[blk],
                                  s3).start()
            row = pl.ds(i, 1)
            pltpu.make_async_copy(vf_t.at[pl.ds(pos_s[i], 1)], pf_o.at[row],
                                  s4).start()
            pltpu.make_async_copy(vf_t.at[pl.ds(neg_s[i], 1)], nf_o.at[row],
                                  s5).start()
            return carry

        lax.fori_loop(0, B, lp, 0, unroll=8)
        for src, dst, sem in ((ue_t, ue_o, s0), (ie_t, pe_o, s1),
                              (ie_t, ne_o, s2), (uv_t, uve_o, s3)):
            pltpu.make_async_copy(src.at[pl.ds(0, 4 * B)], dst, sem).wait()
        for src, dst, sem in ((vf_t, pf_o, s4), (vf_t, nf_o, s5)):
            pltpu.make_async_copy(src.at[pl.ds(0, B)], dst, sem).wait()

    anyspec = pl.BlockSpec(memory_space=pl.ANY)
    smemspec = pl.BlockSpec(memory_space=pltpu.SMEM)
    return pl.pallas_call(
        body,
        in_specs=[smemspec, smemspec, smemspec,
                  anyspec, anyspec, anyspec, anyspec],
        out_shape=[
            jax.ShapeDtypeStruct((4 * B, ED), jnp.float32),
            jax.ShapeDtypeStruct((4 * B, ED), jnp.float32),
            jax.ShapeDtypeStruct((4 * B, ED), jnp.float32),
            jax.ShapeDtypeStruct((4 * B, ED), jnp.float32),
            jax.ShapeDtypeStruct((B, VD), jnp.float32),
            jax.ShapeDtypeStruct((B, VD), jnp.float32),
        ],
        scratch_shapes=[pltpu.SemaphoreType.DMA] * 6,
    )(users, pos_items, neg_items, user_emb, item_emb, uv_emb, v_feat)


def _tc_score(ue4, pe4, ne4, uve4, wu, wp, wn, pbr, nbr, pf, nf, E_w, vb):
    def body(ue_r, pe_r, ne_r, uve_r, wu_r, wp_r, wn_r,
             pbr_r, nbr_r, pf_r, nf_r, ew_r, vb_r,
             pos_o, neg_o, sp_s, sn_s, tp_s, tn_s):
        i = pl.program_id(0)

        @pl.when(i == 0)
        def _():
            def sel(x_r, w_r):
                x = x_r[...]
                w = w_r[...]
                acc = x[:, 0:ED] * w[:, 0:1]
                for kk in range(1, 4):
                    acc = acc + x[:, kk * ED:(kk + 1) * ED] * w[:, kk:kk + 1]
                return acc

            ue = sel(ue_r, wu_r)
            pe = sel(pe_r, wp_r)
            ne = sel(ne_r, wn_r)
            uve = sel(uve_r, wu_r)
            ew = ew_r[...]
            dn = (((1,), (1,)), ((), ()))
            pE = lax.dot_general(pf_r[...], ew, dn,
                                 preferred_element_type=jnp.float32)
            nE = lax.dot_general(nf_r[...], ew, dn,
                                 preferred_element_type=jnp.float32)
            mpos = ue * pe + uve * pE
            mneg = ue * ne + uve * nE
            ones_row = jnp.ones((1, ED), jnp.float32)
            sp = lax.dot_general(ones_row, mpos, dn,
                                 preferred_element_type=jnp.float32)
            sn = lax.dot_general(ones_row, mneg, dn,
                                 preferred_element_type=jnp.float32)
            sp_s[...] = sp + pbr_r[...]
            sn_s[...] = sn + nbr_r[...]
            tp_s[...] = jnp.dot(pf_r[...], vb_r[...],
                                preferred_element_type=jnp.float32)
            tn_s[...] = jnp.dot(nf_r[...], vb_r[...],
                                preferred_element_type=jnp.float32)

        pos_o[...] = sp_s[...] + tp_s[pl.ds(i * BLK, BLK), :]
        neg_o[...] = sn_s[...] + tn_s[pl.ds(i * BLK, BLK), :]

    def full(shape):
        return pl.BlockSpec(shape, lambda i: (0, 0))

    return pl.pallas_call(
        body,
        grid=(NBLK,),
        in_specs=[
            full((B, 4 * ED)), full((B, 4 * ED)), full((B, 4 * ED)),
            full((B, 4 * ED)),
            full((B, 4)), full((B, 4)), full((B, 4)),
            full((1, B)), full((1, B)),
            full((B, VD)), full((B, VD)),
            full((ED, VD)), full((VD, 1)),
        ],
        out_specs=[
            pl.BlockSpec((BLK, B), lambda i: (i, 0)),
            pl.BlockSpec((BLK, B), lambda i: (i, 0)),
        ],
        out_shape=[
            jax.ShapeDtypeStruct((B, B), jnp.float32),
            jax.ShapeDtypeStruct((B, B), jnp.float32),
        ],
        scratch_shapes=[
            pltpu.VMEM((1, B), jnp.float32),
            pltpu.VMEM((1, B), jnp.float32),
            pltpu.VMEM((B, 1), jnp.float32),
            pltpu.VMEM((B, 1), jnp.float32),
        ],
    )(ue4, pe4, ne4, uve4, wu, wp, wn, pbr, nbr, pf, nf, E_w, vb)


def kernel(users, pos_items, neg_items, user_emb, item_emb,
           user_visual_emb, item_bias, visual_bias, E_w, v_feat):
    bias2 = jnp.pad(item_bias.reshape(-1),
                    (0, BIAS_ROWS * 128 - item_bias.shape[0])
                    ).reshape(BIAS_ROWS, 128)
    pb, nb = _sc_bias_gather(pos_items, neg_items, bias2)
    ue4, pe4, ne4, uve4, pf, nf = _tc_gather(
        users, pos_items, neg_items, user_emb, item_emb, user_visual_emb,
        v_feat)
    ue4 = ue4.reshape(B, 4 * ED)
    pe4 = pe4.reshape(B, 4 * ED)
    ne4 = ne4.reshape(B, 4 * ED)
    uve4 = uve4.reshape(B, 4 * ED)
    k4 = jnp.arange(4, dtype=jnp.int32)[None, :]
    wu = (k4 == (users[:, None] & 3)).astype(jnp.float32)
    wp = (k4 == (pos_items[:, None] & 3)).astype(jnp.float32)
    wn = (k4 == (neg_items[:, None] & 3)).astype(jnp.float32)
    pbr = pb.reshape(1, B)
    nbr = nb.reshape(1, B)
    pos, neg = _tc_score(ue4, pe4, ne4, uve4, wu, wp, wn,
                         pbr, nbr, pf, nf, E_w, visual_bias)
    return pos, neg


# speedup vs baseline: 1.0548x; 1.0548x over previous
import functools

import jax
import jax.numpy as jnp
from jax import lax
from jax.experimental import pallas as pl
from jax.experimental.pallas import tpu as pltpu
from jax.experimental.pallas import tpu_sc as plsc

B = 1024
ED = 16
VD = 64
NC = 2
NS = 16
NW = NC * NS
BPW = B // NW
BLK = 128
NBLK = B // BLK
BIAS_ROWS = 7813


def _sc_bias_gather(pos_items, neg_items, bias2):
    mesh = plsc.VectorSubcoreMesh(core_axis_name="c", subcore_axis_name="s")
    out_type = (
        jax.ShapeDtypeStruct((B,), jnp.float32),
        jax.ShapeDtypeStruct((B,), jnp.float32),
    )
    scratch_types = [
        pltpu.VMEM((BPW,), jnp.int32),
        pltpu.VMEM((BPW,), jnp.int32),
        pltpu.VMEM((BPW,), jnp.int32),
        pltpu.VMEM((BPW,), jnp.int32),
        pltpu.VMEM((BPW, 128), jnp.float32),
        pltpu.VMEM((BPW, 128), jnp.float32),
        pltpu.VMEM((BPW,), jnp.float32),
        pltpu.VMEM((BPW,), jnp.float32),
        pltpu.SemaphoreType.DMA,
        pltpu.SemaphoreType.DMA,
    ]

    @functools.partial(pl.kernel, mesh=mesh, out_type=out_type,
                       scratch_types=scratch_types,
                       compiler_params=pltpu.CompilerParams(
                           needs_layout_passes=False))
    def k(pos_h, neg_h, ib_t, pb_o, nb_o,
          p_i, n_i, pbg, nbg, pb_g, nb_g, pb_c, nb_c, gsem, osem):
        wid = lax.axis_index("s") * NC + lax.axis_index("c")
        base = wid * BPW
        pltpu.sync_copy(pos_h.at[pl.ds(base, BPW)], p_i)
        pltpu.sync_copy(neg_h.at[pl.ds(base, BPW)], n_i)
        for gidx in range(BPW // 16):
            sl = pl.ds(gidx * 16, 16)
            pbg[sl] = lax.shift_right_logical(p_i[sl], 7)
            nbg[sl] = lax.shift_right_logical(n_i[sl], 7)
        g1 = pltpu.async_copy(ib_t.at[pbg], pb_g, gsem)
        g2 = pltpu.async_copy(ib_t.at[nbg], nb_g, gsem)
        g1.wait()
        g2.wait()
        for gidx in range(BPW // 16):
            rows16 = pl.ds(gidx * 16, 16)
            rows = lax.iota(jnp.int32, 16) + gidx * 16
            pb_c[rows16] = plsc.load_gather(pb_g, [rows, p_i[rows16] & 127])
            nb_c[rows16] = plsc.load_gather(nb_g, [rows, n_i[rows16] & 127])
        s1 = pltpu.async_copy(pb_c, pb_o.at[pl.ds(base, BPW)], osem)
        s2 = pltpu.async_copy(nb_c, nb_o.at[pl.ds(base, BPW)], osem)
        s1.wait()
        s2.wait()

    return k(pos_items, neg_items, bias2)


def _tc_gather(users, pos_items, neg_items, user_emb, item_emb, uv_emb,
               v_feat):
    def body(users_s, pos_s, neg_s, ue_t, ie_t, uv_t, vf_t,
             ue_o, pe_o, ne_o, uve_o, pf_o, nf_o,
             s0, s1, s2, s3, s4, s5):
        def lp(i, carry):
            u = users_s[i]
            p = pos_s[i]
            n = neg_s[i]
            row = pl.ds(i, 1)
            pltpu.make_async_copy(ue_t.at[pl.ds(u, 1)], ue_o.at[row],
                                  s0).start()
            pltpu.make_async_copy(ie_t.at[pl.ds(p, 1)], pe_o.at[row],
                                  s1).start()
            pltpu.make_async_copy(ie_t.at[pl.ds(n, 1)], ne_o.at[row],
                                  s2).start()
            pltpu.make_async_copy(uv_t.at[pl.ds(u, 1)], uve_o.at[row],
                                  s3).start()
            pltpu.make_async_copy(vf_t.at[pl.ds(p, 1)], pf_o.at[row],
                                  s4).start()
            pltpu.make_async_copy(vf_t.at[pl.ds(n, 1)], nf_o.at[row],
                                  s5).start()
            return carry

        lax.fori_loop(0, B, lp, 0, unroll=8)
        for src, dst, sem in ((ue_t, ue_o, s0), (ie_t, pe_o, s1),
                              (ie_t, ne_o, s2), (uv_t, uve_o, s3),
                              (vf_t, pf_o, s4), (vf_t, nf_o, s5)):
            pltpu.make_async_copy(src.at[pl.ds(0, B)], dst, sem).wait()

    anyspec = pl.BlockSpec(memory_space=pl.ANY)
    smemspec = pl.BlockSpec(memory_space=pltpu.SMEM)
    return pl.pallas_call(
        body,
        in_specs=[smemspec, smemspec, smemspec,
                  anyspec, anyspec, anyspec, anyspec],
        out_shape=[
            jax.ShapeDtypeStruct((B, ED), jnp.float32),
            jax.ShapeDtypeStruct((B, ED), jnp.float32),
            jax.ShapeDtypeStruct((B, ED), jnp.float32),
            jax.ShapeDtypeStruct((B, ED), jnp.float32),
            jax.ShapeDtypeStruct((B, VD), jnp.float32),
            jax.ShapeDtypeStruct((B, VD), jnp.float32),
        ],
        scratch_shapes=[pltpu.SemaphoreType.DMA] * 6,
    )(users, pos_items, neg_items, user_emb, item_emb, uv_emb, v_feat)


def _tc_score(ue, pe, ne, uve, pbr, nbr, pf, nf, E_w, vb):
    def body(ue_r, pe_r, ne_r, uve_r,
             pbr_r, nbr_r, pf_r, nf_r, ew_r, vb_r,
             pos_o, neg_o, sp_s, sn_s, tp_s, tn_s):
        i = pl.program_id(0)

        @pl.when(i == 0)
        def _():
            ue = ue_r[...]
            pe = pe_r[...]
            ne = ne_r[...]
            uve = uve_r[...]
            ew = ew_r[...]
            dn = (((1,), (1,)), ((), ()))
            pE = lax.dot_general(pf_r[...], ew, dn,
                                 preferred_element_type=jnp.float32)
            nE = lax.dot_general(nf_r[...], ew, dn,
                                 preferred_element_type=jnp.float32)
            mpos = ue * pe + uve * pE
            mneg = ue * ne + uve * nE
            ones_row = jnp.ones((1, ED), jnp.float32)
            sp = lax.dot_general(ones_row, mpos, dn,
                                 preferred_element_type=jnp.float32)
            sn = lax.dot_general(ones_row, mneg, dn,
                                 preferred_element_type=jnp.float32)
            sp_s[...] = sp + pbr_r[...]
            sn_s[...] = sn + nbr_r[...]
            tp_s[...] = jnp.dot(pf_r[...], vb_r[...],
                                preferred_element_type=jnp.float32)
            tn_s[...] = jnp.dot(nf_r[...], vb_r[...],
                                preferred_element_type=jnp.float32)

        pos_o[...] = sp_s[...] + tp_s[pl.ds(i * BLK, BLK), :]
        neg_o[...] = sn_s[...] + tn_s[pl.ds(i * BLK, BLK), :]

    def full(shape):
        return pl.BlockSpec(shape, lambda i: (0, 0))

    return pl.pallas_call(
        body,
        grid=(NBLK,),
        in_specs=[
            full((B, ED)), full((B, ED)), full((B, ED)), full((B, ED)),
            full((1, B)), full((1, B)),
            full((B, VD)), full((B, VD)),
            full((ED, VD)), full((VD, 1)),
        ],
        out_specs=[
            pl.BlockSpec((BLK, B), lambda i: (i, 0)),
            pl.BlockSpec((BLK, B), lambda i: (i, 0)),
        ],
        out_shape=[
            jax.ShapeDtypeStruct((B, B), jnp.float32),
            jax.ShapeDtypeStruct((B, B), jnp.float32),
        ],
        scratch_shapes=[
            pltpu.VMEM((1, B), jnp.float32),
            pltpu.VMEM((1, B), jnp.float32),
            pltpu.VMEM((B, 1), jnp.float32),
            pltpu.VMEM((B, 1), jnp.float32),
        ],
    )(ue, pe, ne, uve, pbr, nbr, pf, nf, E_w, vb)


def kernel(users, pos_items, neg_items, user_emb, item_emb,
           user_visual_emb, item_bias, visual_bias, E_w, v_feat):
    bias2 = jnp.pad(item_bias.reshape(-1),
                    (0, BIAS_ROWS * 128 - item_bias.shape[0])
                    ).reshape(BIAS_ROWS, 128)
    pb, nb = _sc_bias_gather(pos_items, neg_items, bias2)
    ue, pe, ne, uve, pf, nf = _tc_gather(
        users, pos_items, neg_items, user_emb, item_emb, user_visual_emb,
        v_feat)
    pbr = pb.reshape(1, B)
    nbr = nb.reshape(1, B)
    pos, neg = _tc_score(ue, pe, ne, uve, pbr, nbr, pf, nf, E_w, visual_bias)
    return pos, neg


# speedup vs baseline: 1.0708x; 1.0151x over previous
import functools

import jax
import jax.numpy as jnp
from jax import lax
from jax.experimental import pallas as pl
from jax.experimental.pallas import tpu as pltpu
from jax.experimental.pallas import tpu_sc as plsc

B = 1024
ED = 16
VD = 64
NC = 2
NS = 16
NW = NC * NS
BPW = B // NW
BLK = 128
NBLK = B // BLK
BIAS_ROWS = 7813


def _sc_gather(users, pos_items, neg_items, user_emb, item_emb, uv_emb,
               bias2):
    mesh = plsc.VectorSubcoreMesh(core_axis_name="c", subcore_axis_name="s")
    out_type = (
        jax.ShapeDtypeStruct((B, ED), jnp.float32),
        jax.ShapeDtypeStruct((B, ED), jnp.float32),
        jax.ShapeDtypeStruct((B, ED), jnp.float32),
        jax.ShapeDtypeStruct((B, ED), jnp.float32),
        jax.ShapeDtypeStruct((B,), jnp.float32),
        jax.ShapeDtypeStruct((B,), jnp.float32),
    )
    scratch_types = [
        pltpu.VMEM((BPW,), jnp.int32),
        pltpu.VMEM((BPW,), jnp.int32),
        pltpu.VMEM((BPW,), jnp.int32),
        pltpu.VMEM((BPW,), jnp.int32),
        pltpu.VMEM((BPW,), jnp.int32),
        pltpu.VMEM((BPW, ED), jnp.float32),
        pltpu.VMEM((BPW, ED), jnp.float32),
        pltpu.VMEM((BPW, ED), jnp.float32),
        pltpu.VMEM((BPW, ED), jnp.float32),
        pltpu.VMEM((BPW, 128), jnp.float32),
        pltpu.VMEM((BPW, 128), jnp.float32),
        pltpu.VMEM((BPW,), jnp.float32),
        pltpu.VMEM((BPW,), jnp.float32),
        pltpu.SemaphoreType.DMA,
        pltpu.SemaphoreType.DMA,
    ]

    @functools.partial(pl.kernel, mesh=mesh, out_type=out_type,
                       scratch_types=scratch_types,
                       compiler_params=pltpu.CompilerParams(
                           needs_layout_passes=False))
    def k(users_h, pos_h, neg_h, ue_t, ie_t, uv_t, ib_t,
          ue_o, pe_o, ne_o, uve_o, pb_o, nb_o,
          u_i, p_i, n_i, pbg, nbg,
          ue_b, pe_b, ne_b, uve_b, pb_g, nb_g, pb_c, nb_c, gsem, osem):
        wid = lax.axis_index("s") * NC + lax.axis_index("c")
        base = wid * BPW
        pltpu.sync_copy(users_h.at[pl.ds(base, BPW)], u_i)
        pltpu.sync_copy(pos_h.at[pl.ds(base, BPW)], p_i)
        pltpu.sync_copy(neg_h.at[pl.ds(base, BPW)], n_i)
        for gidx in range(BPW // 16):
            sl = pl.ds(gidx * 16, 16)
            pbg[sl] = lax.shift_right_logical(p_i[sl], 7)
            nbg[sl] = lax.shift_right_logical(n_i[sl], 7)
        pltpu.async_copy(ib_t.at[pbg], pb_g, gsem)
        pltpu.async_copy(ib_t.at[nbg], nb_g, gsem)
        for gidx in range(BPW // 16):
            uvv = u_i[pl.ds(gidx * 16, 16)]
            pvv = p_i[pl.ds(gidx * 16, 16)]
            nvv = n_i[pl.ds(gidx * 16, 16)]
            for j in range(16):
                i = gidx * 16 + j
                u = uvv[j]
                p = pvv[j]
                n = nvv[j]
                row = pl.ds(i, 1)
                pltpu.async_copy(ue_t.at[pl.ds(u, 1), :], ue_b.at[row, :],
                                 gsem)
                pltpu.async_copy(ie_t.at[pl.ds(p, 1), :], pe_b.at[row, :],
                                 gsem)
                pltpu.async_copy(ie_t.at[pl.ds(n, 1), :], ne_b.at[row, :],
                                 gsem)
                pltpu.async_copy(uv_t.at[pl.ds(u, 1), :], uve_b.at[row, :],
                                 gsem)
        pltpu.make_async_copy(ib_t.at[pl.ds(0, BPW)], pb_g, gsem).wait()
        pltpu.make_async_copy(ib_t.at[pl.ds(0, BPW)], nb_g, gsem).wait()
        for src, buf in ((ue_t, ue_b), (ie_t, pe_b), (ie_t, ne_b),
                         (uv_t, uve_b)):
            pltpu.make_async_copy(src.at[pl.ds(0, BPW)], buf, gsem).wait()
        for gidx in range(BPW // 16):
            rows16 = pl.ds(gidx * 16, 16)
            rows = lax.iota(jnp.int32, 16) + gidx * 16
            pb_c[rows16] = plsc.load_gather(pb_g, [rows, p_i[rows16] & 127])
            nb_c[rows16] = plsc.load_gather(nb_g, [rows, n_i[rows16] & 127])
        stores = []
        for buf, out in ((ue_b, ue_o), (pe_b, pe_o), (ne_b, ne_o),
                         (uve_b, uve_o), (pb_c, pb_o), (nb_c, nb_o)):
            stores.append(pltpu.async_copy(buf, out.at[pl.ds(base, BPW)],
                                           osem))
        for st in stores:
            st.wait()

    return k(users, pos_items, neg_items, user_emb, item_emb, uv_emb, bias2)


def _tc_gather(pos_items, neg_items, v_feat):
    def body(pos_s, neg_s, vf_t, pf_o, nf_o, s4, s5):
        def lp(i, carry):
            row = pl.ds(i, 1)
            pltpu.make_async_copy(vf_t.at[pl.ds(pos_s[i], 1)], pf_o.at[row],
                                  s4).start()
            pltpu.make_async_copy(vf_t.at[pl.ds(neg_s[i], 1)], nf_o.at[row],
                                  s5).start()
            return carry

        lax.fori_loop(0, B, lp, 0, unroll=8)
        for dst, sem in ((pf_o, s4), (nf_o, s5)):
            pltpu.make_async_copy(vf_t.at[pl.ds(0, B)], dst, sem).wait()

    anyspec = pl.BlockSpec(memory_space=pl.ANY)
    smemspec = pl.BlockSpec(memory_space=pltpu.SMEM)
    return pl.pallas_call(
        body,
        in_specs=[smemspec, smemspec, anyspec],
        out_shape=[
            jax.ShapeDtypeStruct((B, VD), jnp.float32),
            jax.ShapeDtypeStruct((B, VD), jnp.float32),
        ],
        scratch_shapes=[pltpu.SemaphoreType.DMA] * 2,
    )(pos_items, neg_items, v_feat)


def _tc_score(ue, pe, ne, uve, pbr, nbr, pf, nf, E_w, vb):
    def body(ue_r, pe_r, ne_r, uve_r,
             pbr_r, nbr_r, pf_r, nf_r, ew_r, vb_r,
             pos_o, neg_o, sp_s, sn_s, tp_s, tn_s):
        i = pl.program_id(0)

        @pl.when(i == 0)
        def _():
            ue = ue_r[...]
            pe = pe_r[...]
            ne = ne_r[...]
            uve = uve_r[...]
            ew = ew_r[...]
            dn = (((1,), (1,)), ((), ()))
            pE = lax.dot_general(pf_r[...], ew, dn,
                                 preferred_element_type=jnp.float32)
            nE = lax.dot_general(nf_r[...], ew, dn,
                                 preferred_element_type=jnp.float32)
            mpos = ue * pe + uve * pE
            mneg = ue * ne + uve * nE
            ones_row = jnp.ones((1, ED), jnp.float32)
            sp = lax.dot_general(ones_row, mpos, dn,
                                 preferred_element_type=jnp.float32)
            sn = lax.dot_general(ones_row, mneg, dn,
                                 preferred_element_type=jnp.float32)
            sp_s[...] = sp + pbr_r[...]
            sn_s[...] = sn + nbr_r[...]
            tp_s[...] = jnp.dot(pf_r[...], vb_r[...],
                                preferred_element_type=jnp.float32)
            tn_s[...] = jnp.dot(nf_r[...], vb_r[...],
                                preferred_element_type=jnp.float32)

        pos_o[...] = sp_s[...] + tp_s[pl.ds(i * BLK, BLK), :]
        neg_o[...] = sn_s[...] + tn_s[pl.ds(i * BLK, BLK), :]

    def full(shape):
        return pl.BlockSpec(shape, lambda i: (0, 0))

    return pl.pallas_call(
        body,
        grid=(NBLK,),
        in_specs=[
            full((B, ED)), full((B, ED)), full((B, ED)), full((B, ED)),
            full((1, B)), full((1, B)),
            full((B, VD)), full((B, VD)),
            full((ED, VD)), full((VD, 1)),
        ],
        out_specs=[
            pl.BlockSpec((BLK, B), lambda i: (i, 0)),
            pl.BlockSpec((BLK, B), lambda i: (i, 0)),
        ],
        out_shape=[
            jax.ShapeDtypeStruct((B, B), jnp.float32),
            jax.ShapeDtypeStruct((B, B), jnp.float32),
        ],
        scratch_shapes=[
            pltpu.VMEM((1, B), jnp.float32),
            pltpu.VMEM((1, B), jnp.float32),
            pltpu.VMEM((B, 1), jnp.float32),
            pltpu.VMEM((B, 1), jnp.float32),
        ],
    )(ue, pe, ne, uve, pbr, nbr, pf, nf, E_w, vb)


def kernel(users, pos_items, neg_items, user_emb, item_emb,
           user_visual_emb, item_bias, visual_bias, E_w, v_feat):
    bias2 = jnp.pad(item_bias.reshape(-1),
                    (0, BIAS_ROWS * 128 - item_bias.shape[0])
                    ).reshape(BIAS_ROWS, 128)
    ue, pe, ne, uve, pb, nb = _sc_gather(
        users, pos_items, neg_items, user_emb, item_emb, user_visual_emb,
        bias2)
    pf, nf = _tc_gather(pos_items, neg_items, v_feat)
    pbr = pb.reshape(1, B)
    nbr = nb.reshape(1, B)
    pos, neg = _tc_score(ue, pe, ne, uve, pbr, nbr, pf, nf, E_w, visual_bias)
    return pos, neg
